# baseline (speedup 1.0000x reference)
import functools

import jax
import jax.numpy as jnp
from jax import lax
from jax.experimental import pallas as pl
from jax.experimental.pallas import tpu as pltpu
from jax.experimental.pallas import tpu_sc as plsc

B, H, W = 8, 384, 512
HW = H * W
NUM_TOP = 16384

NC, NS = 2, 16
NW = NC * NS
CHUNKS_PER_B = NW // B
CHUNK = NUM_TOP // CHUNKS_PER_B
ROWS = CHUNK // 128
LANES = 16
VITERS = 128 // LANES


NSEM = 4
GROUP = ROWS // NSEM


def _sc_body(depth_hbm, coeff_hbm, gidx_hbm, out_hbm, idx_v, d_v, coeff_v,
             out_v, sem0, sem1, sem2, sem3):
    sems = (sem0, sem1, sem2, sem3)
    c = lax.axis_index("c")
    s = lax.axis_index("s")
    wid = s * NC + c
    b = wid // CHUNKS_PER_B
    ch = lax.rem(wid, CHUNKS_PER_B)

    pltpu.sync_copy(gidx_hbm.at[b, ch], idx_v)
    pltpu.sync_copy(coeff_hbm.at[b], coeff_v)

    for g in range(NSEM):
        sem_g = sems[g]

        def fire(j, carry, sem_g=sem_g):
            pltpu.async_copy(depth_hbm.at[idx_v.at[j]], d_v.at[j], sem_g)
            return carry

        lax.fori_loop(g * GROUP, (g + 1) * GROUP, fire, 0)

    k00 = coeff_v[0]
    k01 = coeff_v[1]
    k02 = coeff_v[2]
    k10 = coeff_v[3]
    k11 = coeff_v[4]
    k12 = coeff_v[5]
    k20 = coeff_v[6]
    k21 = coeff_v[7]
    k22 = coeff_v[8]
    ones = jnp.full((LANES,), 1.0, dtype=jnp.float32)

    def row_body(j, carry):
        for l in range(VITERS):
            ii = l * LANES
            gi = idx_v[j, pl.ds(ii, LANES)]
            d = d_v[j, pl.ds(ii, LANES)]
            x = (gi & (W - 1)).astype(jnp.float32)
            y = (gi >> 9).astype(jnp.float32)
            off = j * 128 + ii
            out_v[0, pl.ds(off, LANES)] = d * (k00 * x + k01 * y + k02)
            out_v[1, pl.ds(off, LANES)] = d * (k10 * x + k11 * y + k12)
            out_v[2, pl.ds(off, LANES)] = d * (k20 * x + k21 * y + k22)
            out_v[3, pl.ds(off, LANES)] = ones
        return carry

    for g in range(NSEM):
        sem_g = sems[g]

        def drain(j, carry, sem_g=sem_g):
            pltpu.make_async_copy(
                depth_hbm.at[idx_v.at[j]], d_v.at[j], sem_g).wait()
            return carry

        lax.fori_loop(g * GROUP, (g + 1) * GROUP, drain, 0)
        lax.fori_loop(g * GROUP, (g + 1) * GROUP, row_body, 0)

    pltpu.sync_copy(out_v, out_hbm.at[b, :, pl.ds(ch * CHUNK, CHUNK)])


@jax.jit
def _backproject(depth, inv_K, top_k_indices):
    depth_flat = jnp.maximum(depth.reshape(B * HW), jnp.float32(0.0))
    base = (jnp.arange(B, dtype=jnp.int32) * HW)[:, None]
    gidx = (top_k_indices + base).reshape(B, CHUNKS_PER_B, ROWS, 128)

    A = inv_K[:, :3, :3]
    brow = (jnp.arange(B, dtype=jnp.float32) * float(H))[:, None]
    c2 = A[:, :, 2] - A[:, :, 1] * brow
    coeff = jnp.stack([A[:, :, 0], A[:, :, 1], c2], axis=-1).reshape(B, 9)
    coeff16 = jnp.broadcast_to(coeff[:, :, None], (B, 9, LANES))

    run = pl.kernel(
        _sc_body,
        out_type=jax.ShapeDtypeStruct((B, 4, NUM_TOP), jnp.float32),
        mesh=plsc.VectorSubcoreMesh(core_axis_name="c", subcore_axis_name="s"),
        scratch_types=[
            pltpu.VMEM((ROWS, 128), jnp.int32),
            pltpu.VMEM((ROWS, 128), jnp.float32),
            pltpu.VMEM((9, LANES), jnp.float32),
            pltpu.VMEM((4, CHUNK), jnp.float32),
            pltpu.SemaphoreType.DMA,
            pltpu.SemaphoreType.DMA,
            pltpu.SemaphoreType.DMA,
            pltpu.SemaphoreType.DMA,
        ],
    )
    return run(depth_flat, coeff16, gidx)


def kernel(depth, inv_K, top_k_indices):
    return _backproject(depth, inv_K, top_k_indices)


# speedup vs baseline: 1.4258x; 1.4258x over previous
import jax
import jax.numpy as jnp
from jax import lax
from jax.experimental import pallas as pl
from jax.experimental.pallas import tpu as pltpu
from jax.experimental.pallas import tpu_sc as plsc

B, H, W = 8, 384, 512
HW = H * W
NUM_TOP = 16384

NC, NS = 2, 16
NW = NC * NS
CHUNKS_PER_B = NW // B
CHUNK = NUM_TOP // CHUNKS_PER_B
ROWS = CHUNK // 128
LANES = 16
VITERS = 128 // LANES


def _sc_body(depth_hbm, coeff_hbm, sidx_hbm, out_hbm, idx_v, d_v, coeff_v,
             out_v, sem):
    c = lax.axis_index("c")
    s = lax.axis_index("s")
    wid = s * NC + c
    b = wid // CHUNKS_PER_B
    ch = lax.rem(wid, CHUNKS_PER_B)

    pltpu.sync_copy(sidx_hbm.at[b, ch], idx_v)
    pltpu.sync_copy(coeff_hbm.at[b], coeff_v)

    def fire(j, carry):
        pltpu.async_copy(depth_hbm.at[idx_v.at[j]], d_v.at[j], sem)
        return carry

    lax.fori_loop(0, ROWS, fire, 0)

    def drain(j, carry):
        pltpu.make_async_copy(depth_hbm.at[idx_v.at[j]], d_v.at[j], sem).wait()
        return carry

    lax.fori_loop(0, ROWS, drain, 0)

    k00 = coeff_v[0]
    k01 = coeff_v[1]
    k02 = coeff_v[2]
    k10 = coeff_v[3]
    k11 = coeff_v[4]
    k12 = coeff_v[5]
    k20 = coeff_v[6]
    k21 = coeff_v[7]
    k22 = coeff_v[8]
    ones = jnp.full((LANES,), 1.0, dtype=jnp.float32)

    def row_body(j, carry):
        for l in range(VITERS):
            ii = l * LANES
            si = idx_v[j, pl.ds(ii, LANES)]
            d = d_v[j, pl.ds(ii, LANES)]
            x = ((si & 127) | ((si & 0xC00) >> 3)).astype(jnp.float32)
            yg = (((si & ~4095) >> 9) | ((si & 0x380) >> 7)).astype(jnp.float32)
            off = j * 128 + ii
            out_v[0, pl.ds(off, LANES)] = d * (k00 * x + k01 * yg + k02)
            out_v[1, pl.ds(off, LANES)] = d * (k10 * x + k11 * yg + k12)
            out_v[2, pl.ds(off, LANES)] = d * (k20 * x + k21 * yg + k22)
            out_v[3, pl.ds(off, LANES)] = ones
        return carry

    lax.fori_loop(0, ROWS, row_body, 0)

    pltpu.sync_copy(out_v, out_hbm.at[b, :, pl.ds(ch * CHUNK, CHUNK)])


@jax.jit
def _backproject(depth, inv_K, top_k_indices):
    depth_lin = depth.reshape(B, H // 8, 8, W // 128, 128)
    depth_lin = depth_lin.transpose(0, 1, 3, 2, 4).reshape(B * HW)

    base = (jnp.arange(B, dtype=jnp.int32) * HW)[:, None]
    g = top_k_indices + base
    sidx = ((g & ~4095) | ((g & 0x180) << 3) | ((g & 0xE00) >> 2) | (g & 127))
    sidx = sidx.reshape(B, CHUNKS_PER_B, ROWS, 128)

    A = inv_K[:, :3, :3]
    brow = (jnp.arange(B, dtype=jnp.float32) * float(H))[:, None]
    c2 = A[:, :, 2] - A[:, :, 1] * brow
    coeff = jnp.stack([A[:, :, 0], A[:, :, 1], c2], axis=-1).reshape(B, 9)
    coeff16 = jnp.broadcast_to(coeff[:, :, None], (B, 9, LANES))

    run = pl.kernel(
        _sc_body,
        out_type=jax.ShapeDtypeStruct((B, 4, NUM_TOP), jnp.float32),
        mesh=plsc.VectorSubcoreMesh(core_axis_name="c", subcore_axis_name="s"),
        scratch_types=[
            pltpu.VMEM((ROWS, 128), jnp.int32),
            pltpu.VMEM((ROWS, 128), jnp.float32),
            pltpu.VMEM((9, LANES), jnp.float32),
            pltpu.VMEM((4, CHUNK), jnp.float32),
            pltpu.SemaphoreType.DMA,
        ],
    )
    return run(depth_lin, coeff16, sidx)


def kernel(depth, inv_K, top_k_indices):
    return _backproject(depth, inv_K, top_k_indices)
